# Initial kernel scaffold; baseline (speedup 1.0000x reference)
import jax
import jax.numpy as jnp
from jax.experimental import pallas as pl


def kernel(x, edge_index, W1a, b1a, W2a, b2a, W1b, b1b, W2b, b2b):
    raise NotImplementedError("write your pallas kernel here")


# speedup vs baseline: 3.2244x; 3.2244x over previous
import functools

import jax
import jax.numpy as jnp
from jax import lax
from jax.experimental import pallas as pl
from jax.experimental.pallas import tpu as pltpu
from jax.experimental.pallas import tpu_sc as plsc

N_NODES = 10000
N_EDGES = 320000
D = 128

NC = 2
NS = 16
TILES = NC * NS
K = 128
CH = 80
E_PAD = TILES * CH * K
N_PAD = 10112
ZROWS = N_PAD // NS

_mesh = plsc.VectorSubcoreMesh(
    core_axis_name="c", subcore_axis_name="s", num_cores=NC, num_subcores=NS)


@functools.partial(
    pl.kernel,
    out_type=jax.ShapeDtypeStruct((NC, N_PAD, D), jnp.float32),
    mesh=_mesh,
    scratch_types=[
        pltpu.VMEM((CH, K), jnp.int32),
        pltpu.VMEM((CH, K), jnp.int32),
        pltpu.VMEM((K, D), jnp.float32),
        pltpu.VMEM_SHARED((N_PAD, D), jnp.float32),
        pltpu.SemaphoreType.DMA,
    ],
)
def _agg_call(feat_hbm, src_hbm, dst_hbm, out_hbm, src_v, dst_v, rows_v, agg_sh, sem):
    c = lax.axis_index("c")
    s = lax.axis_index("s")
    tile = c * NS + s

    zero = jnp.zeros((16,), jnp.float32)

    def _zrow(i, carry):
        for j in range(D // 16):
            rows_v[i, pl.ds(j * 16, 16)] = zero
        return carry

    lax.fori_loop(0, K, _zrow, 0)

    zbase = s * ZROWS
    for q in range(ZROWS // K):
        pltpu.sync_copy(rows_v, agg_sh.at[pl.ds(zbase + q * K, K)])
    rem = ZROWS % K
    if rem:
        pltpu.sync_copy(rows_v.at[pl.ds(0, rem)],
                        agg_sh.at[pl.ds(zbase + (ZROWS // K) * K, rem)])

    pltpu.sync_copy(src_hbm.at[pl.ds(tile * CH, CH)], src_v)
    pltpu.sync_copy(dst_hbm.at[pl.ds(tile * CH, CH)], dst_v)

    plsc.subcore_barrier()

    def _chunk(j, carry):
        pltpu.async_copy(feat_hbm.at[src_v.at[j]], rows_v, sem).wait()
        pltpu.sync_copy(rows_v, agg_sh.at[dst_v.at[j]], add=True)
        return carry

    lax.fori_loop(0, CH, _chunk, 0)

    plsc.subcore_barrier()

    pltpu.sync_copy(agg_sh.at[pl.ds(zbase, ZROWS)],
                    out_hbm.at[c, pl.ds(zbase, ZROWS)])


def _mlp_body(x_ref, p0_ref, p1_ref, w1_ref, b1_ref, w2_ref, b2_ref, o_ref,
              *, final_relu):
    a = x_ref[...] + p0_ref[...] + p1_ref[...]
    t = jnp.dot(a, w1_ref[...], preferred_element_type=jnp.float32) + b1_ref[...]
    t = jnp.maximum(t, 0.0)
    o = jnp.dot(t, w2_ref[...], preferred_element_type=jnp.float32) + b2_ref[...]
    if final_relu:
        o = jnp.maximum(o, 0.0)
    o_ref[...] = o


def _mlp_call(x, p0, p1, w1, b1, w2, b2, final_relu):
    bm = 2000
    grid = (N_NODES // bm,)
    row_spec = pl.BlockSpec((bm, D), lambda i: (i, 0))
    full_spec = pl.BlockSpec((D, D), lambda i: (0, 0))
    bias_spec = pl.BlockSpec((1, D), lambda i: (0, 0))
    return pl.pallas_call(
        functools.partial(_mlp_body, final_relu=final_relu),
        grid=grid,
        in_specs=[row_spec, row_spec, row_spec, full_spec, bias_spec,
                  full_spec, bias_spec],
        out_specs=row_spec,
        out_shape=jax.ShapeDtypeStruct((N_NODES, D), jnp.float32),
    )(x, p0, p1, w1, b1.reshape(1, D), w2, b2.reshape(1, D))


def kernel(x, edge_index, W1a, b1a, W2a, b2a, W1b, b1b, W2b, b2b):
    src = edge_index[0].astype(jnp.int32)
    dst = edge_index[1].astype(jnp.int32)
    pad = E_PAD - N_EDGES
    src_p = jnp.concatenate([src, jnp.zeros((pad,), jnp.int32)]).reshape(-1, K)
    dst_p = jnp.concatenate([dst, jnp.full((pad,), N_NODES, jnp.int32)]).reshape(-1, K)

    p = _agg_call(x, src_p, dst_p)
    h = _mlp_call(x, p[0, :N_NODES], p[1, :N_NODES], W1a, b1a, W2a, b2a,
                  final_relu=True)

    p2 = _agg_call(h, src_p, dst_p)
    w2b = jnp.zeros((D, D), jnp.float32).at[:, :W2b.shape[1]].set(W2b)
    b2b = jnp.zeros((D,), jnp.float32).at[:W2b.shape[1]].set(b2b)
    out = _mlp_call(h, p2[0, :N_NODES], p2[1, :N_NODES], W1b, b1b, w2b, b2b,
                    final_relu=False)
    return out[:, :W2b.shape[1]]


# speedup vs baseline: 3.2619x; 1.0116x over previous
import functools

import jax
import jax.numpy as jnp
from jax import lax
from jax.experimental import pallas as pl
from jax.experimental.pallas import tpu as pltpu
from jax.experimental.pallas import tpu_sc as plsc

N_NODES = 10000
N_EDGES = 320000
D = 128

NC = 2
NS = 16
TILES = NC * NS
K = 128
CH = 80
CQ = 16
E_PAD = TILES * CH * K
N_PAD = 10112
ZROWS = N_PAD // NS

_mesh = plsc.VectorSubcoreMesh(
    core_axis_name="c", subcore_axis_name="s", num_cores=NC, num_subcores=NS)


@functools.partial(
    pl.kernel,
    out_type=jax.ShapeDtypeStruct((NC, N_PAD, D), jnp.float32),
    mesh=_mesh,
    scratch_types=[
        pltpu.VMEM((CQ, K), jnp.int32),
        pltpu.VMEM((CQ, K), jnp.int32),
        pltpu.VMEM((K, D), jnp.float32),
        pltpu.VMEM((K, D), jnp.float32),
        pltpu.VMEM_SHARED((N_PAD, D), jnp.float32),
        pltpu.SemaphoreType.DMA,
        pltpu.SemaphoreType.DMA,
    ],
)
def _agg_call(feat_hbm, src_hbm, dst_hbm, out_hbm, src_v, dst_v, rows0_v,
              rows1_v, agg_sh, sem0, sem1):
    c = lax.axis_index("c")
    s = lax.axis_index("s")
    tile = c * NS + s

    zero = jnp.zeros((16,), jnp.float32)

    def _zrow(i, carry):
        for j in range(D // 16):
            rows0_v[i, pl.ds(j * 16, 16)] = zero
        return carry

    lax.fori_loop(0, K, _zrow, 0)

    zbase = s * ZROWS
    for q in range(ZROWS // K):
        pltpu.sync_copy(rows0_v, agg_sh.at[pl.ds(zbase + q * K, K)])
    rem = ZROWS % K
    if rem:
        pltpu.sync_copy(rows0_v.at[pl.ds(0, rem)],
                        agg_sh.at[pl.ds(zbase + (ZROWS // K) * K, rem)])

    plsc.subcore_barrier()

    for q in range(CH // CQ):
        qbase = tile * CH + q * CQ
        pltpu.sync_copy(src_hbm.at[pl.ds(qbase, CQ)], src_v)
        pltpu.sync_copy(dst_hbm.at[pl.ds(qbase, CQ)], dst_v)

        def _chunk(i, carry):
            g = i * 2
            cp_a = pltpu.async_copy(feat_hbm.at[src_v.at[g]], rows0_v, sem0)
            cp_b = pltpu.async_copy(feat_hbm.at[src_v.at[g + 1]], rows1_v, sem1)
            cp_a.wait()
            pltpu.sync_copy(rows0_v, agg_sh.at[dst_v.at[g]], add=True)
            cp_b.wait()
            pltpu.sync_copy(rows1_v, agg_sh.at[dst_v.at[g + 1]], add=True)
            return carry

        lax.fori_loop(0, CQ // 2, _chunk, 0)

    plsc.subcore_barrier()

    pltpu.sync_copy(agg_sh.at[pl.ds(zbase, ZROWS)],
                    out_hbm.at[c, pl.ds(zbase, ZROWS)])


def _mlp_body(x_ref, p0_ref, p1_ref, w1_ref, b1_ref, w2_ref, b2_ref, o_ref,
              *, final_relu):
    a = x_ref[...] + p0_ref[...] + p1_ref[...]
    t = jnp.dot(a, w1_ref[...], preferred_element_type=jnp.float32) + b1_ref[...]
    t = jnp.maximum(t, 0.0)
    o = jnp.dot(t, w2_ref[...], preferred_element_type=jnp.float32) + b2_ref[...]
    if final_relu:
        o = jnp.maximum(o, 0.0)
    o_ref[...] = o


def _mlp_call(x, p0, p1, w1, b1, w2, b2, final_relu):
    bm = 2000
    grid = (N_NODES // bm,)
    row_spec = pl.BlockSpec((bm, D), lambda i: (i, 0))
    full_spec = pl.BlockSpec((D, D), lambda i: (0, 0))
    bias_spec = pl.BlockSpec((1, D), lambda i: (0, 0))
    return pl.pallas_call(
        functools.partial(_mlp_body, final_relu=final_relu),
        grid=grid,
        in_specs=[row_spec, row_spec, row_spec, full_spec, bias_spec,
                  full_spec, bias_spec],
        out_specs=row_spec,
        out_shape=jax.ShapeDtypeStruct((N_NODES, D), jnp.float32),
    )(x, p0, p1, w1, b1.reshape(1, D), w2, b2.reshape(1, D))


def kernel(x, edge_index, W1a, b1a, W2a, b2a, W1b, b1b, W2b, b2b):
    src = edge_index[0].astype(jnp.int32)
    dst = edge_index[1].astype(jnp.int32)
    pad = E_PAD - N_EDGES
    src_p = jnp.concatenate([src, jnp.zeros((pad,), jnp.int32)]).reshape(-1, K)
    dst_p = jnp.concatenate([dst, jnp.full((pad,), N_NODES, jnp.int32)]).reshape(-1, K)

    p = _agg_call(x, src_p, dst_p)
    h = _mlp_call(x, p[0, :N_NODES], p[1, :N_NODES], W1a, b1a, W2a, b2a,
                  final_relu=True)

    p2 = _agg_call(h, src_p, dst_p)
    w2b = jnp.zeros((D, D), jnp.float32).at[:, :W2b.shape[1]].set(W2b)
    b2b = jnp.zeros((D,), jnp.float32).at[:W2b.shape[1]].set(b2b)
    out = _mlp_call(h, p2[0, :N_NODES], p2[1, :N_NODES], W1b, b1b, w2b, b2b,
                    final_relu=False)
    return out[:, :W2b.shape[1]]
